# baseline (speedup 1.0000x reference)
import functools

import jax
import jax.numpy as jnp
from jax.experimental import pallas as pl
from jax.experimental.pallas import tpu as pltpu
from jax.experimental.pallas import tpu_sc as plsc

_BETA = 0.25
_BC = 2048
_GW = 128


def _argmin_body(a_ref, x_ref, cb_ref, b_ref, idx_ref, rmin_ref, raux_ref,
                 *, nt, nc, bc):
    j = pl.program_id(0)
    ng = bc // 128

    @pl.when(j == 0)
    def _():
        rmin_ref[...] = jnp.full((nt, 128), jnp.inf, jnp.float32)
        raux_ref[...] = jnp.zeros((nt, 128), jnp.int32)

    m2 = jax.lax.dot_general(
        x_ref[...], cb_ref[...], (((1,), (1,)), ((), ())),
        precision=jax.lax.Precision.DEFAULT,
        preferred_element_type=jnp.float32)
    a = a_ref[...]
    heap = []
    for g in range(ng):
        sl = slice(g * 128, (g + 1) * 128)
        d = (a - m2[:, sl]) + b_ref[:, sl]
        heap.append((d, j * ng + g))
    while len(heap) > 1:
        nxt = []
        for p in range(0, len(heap), 2):
            (va, ga), (vb, gb) = heap[p], heap[p + 1]
            pred = vb < va
            nxt.append((jnp.where(pred, vb, va), jnp.where(pred, gb, ga)))
        heap = nxt
    val, gid = heap[0]
    pred = val < rmin_ref[...]
    rmin_ref[...] = jnp.where(pred, val, rmin_ref[...])
    raux_ref[...] = jnp.where(pred, gid, raux_ref[...])

    @pl.when(j == nc // bc - 1)
    def _():
        rmin = rmin_ref[...]
        gmin = jnp.min(rmin, axis=1, keepdims=True)
        col = (raux_ref[...] * 128
               + jax.lax.broadcasted_iota(jnp.int32, (nt, 128), 1))
        cand = jnp.where(rmin == gmin, col, nc)
        idx_ref[...] = jnp.min(cand, axis=1, keepdims=True)


def _sc_gather(cb_pad, idx_row, nt):
    mesh = plsc.VectorSubcoreMesh(core_axis_name="core",
                                  subcore_axis_name="subcore")
    w = cb_pad.shape[1]

    @pl.kernel(out_type=jax.ShapeDtypeStruct((nt, w), cb_pad.dtype),
               mesh=mesh)
    def gk(cb_hbm, i_hbm, o_hbm):
        def body(i_vmem, o_vmem):
            pltpu.sync_copy(cb_hbm.at[i_vmem.at[0]], o_vmem)

        pltpu.emit_pipeline(
            body,
            grid=(nt // _GW,),
            in_specs=[pl.BlockSpec((1, _GW), index_map=lambda i: (0, i))],
            out_specs=[pl.BlockSpec((_GW, w), index_map=lambda i: (i, 0))],
            core_axis_name=("core", "subcore"),
            dimension_semantics=(pltpu.PARALLEL,),
        )(i_hbm, o_hbm)

    return gk(cb_pad, idx_row)


def _finish_body(x_ref, q_ref, idx_ref, qst_ref, loss_ref):
    x = x_ref[...]
    d = x_ref.shape[1]
    rem = idx_ref[...] % (128 // d)
    q = q_ref[:, :d]
    for k in range(1, 128 // d):
        q = jnp.where(rem == k, q_ref[:, k * d:(k + 1) * d], q)
    qst_ref[...] = x + (q - x)
    diff = x - q
    msq = jnp.mean(diff * diff)
    loss_ref[...] = jnp.full((1, 1), _BETA * msq + msq, jnp.float32)


def kernel(latent, codebook):
    B, S, D = latent.shape
    nt = B * S
    nc = codebook.shape[0]
    bc = _BC
    flat = latent.reshape(-1, D)
    a = jnp.sum(flat ** 2, axis=1, keepdims=True)
    b = jnp.sum(codebook ** 2, axis=1).reshape(1, nc)
    grid = (nc // bc,)

    idx = pl.pallas_call(
        functools.partial(_argmin_body, nt=nt, nc=nc, bc=bc),
        grid=grid,
        in_specs=[
            pl.BlockSpec((nt, 1), lambda j: (0, 0)),
            pl.BlockSpec((nt, D), lambda j: (0, 0)),
            pl.BlockSpec((bc, D), lambda j: (j, 0)),
            pl.BlockSpec((1, bc), lambda j: (0, j)),
        ],
        out_specs=pl.BlockSpec((nt, 1), lambda j: (0, 0)),
        out_shape=jax.ShapeDtypeStruct((nt, 1), jnp.int32),
        scratch_shapes=[pltpu.VMEM((nt, 128), jnp.float32),
                        pltpu.VMEM((nt, 128), jnp.int32)],
    )(a, 2.0 * flat, codebook, b)

    perrow = 128 // D
    cb_packed = codebook.reshape(nc // perrow, 128)
    q = _sc_gather(cb_packed, (idx // perrow).reshape(1, nt), nt)

    qst, loss = pl.pallas_call(
        _finish_body,
        in_specs=[
            pl.BlockSpec((nt, D), lambda: (0, 0)),
            pl.BlockSpec((nt, 128), lambda: (0, 0)),
            pl.BlockSpec((nt, 1), lambda: (0, 0)),
        ],
        out_specs=[pl.BlockSpec((nt, D), lambda: (0, 0)),
                   pl.BlockSpec((1, 1), lambda: (0, 0))],
        out_shape=[jax.ShapeDtypeStruct((nt, D), jnp.float32),
                   jax.ShapeDtypeStruct((1, 1), jnp.float32)],
    )(flat, q, idx)

    return (qst.reshape(B, S, D), loss.reshape(()), idx.reshape(nt))


# speedup vs baseline: 1.0888x; 1.0888x over previous
import functools

import jax
import jax.numpy as jnp
from jax.experimental import pallas as pl
from jax.experimental.pallas import tpu as pltpu
from jax.experimental.pallas import tpu_sc as plsc

_BETA = 0.25
_BC = 2048
_GW = 128


def _argmin_body(a_ref, x_ref, cb_ref, b_ref, idx_ref, rmin_ref, raux_ref,
                 *, nt, nc, bc):
    j = pl.program_id(0)
    ng = bc // 128

    @pl.when(j == 0)
    def _():
        rmin_ref[...] = jnp.full((nt, 128), jnp.inf, jnp.float32)
        raux_ref[...] = jnp.zeros((nt, 128), jnp.int32)

    m2 = jax.lax.dot_general(
        x_ref[...], cb_ref[...], (((1,), (1,)), ((), ())),
        precision=jax.lax.Precision.DEFAULT,
        preferred_element_type=jnp.float32)
    a = a_ref[...]
    heap = []
    for g in range(ng):
        sl = slice(g * 128, (g + 1) * 128)
        d = (a - m2[:, sl]) + b_ref[:, sl]
        heap.append((d, j * ng + g))
    while len(heap) > 1:
        nxt = []
        for p in range(0, len(heap), 2):
            (va, ga), (vb, gb) = heap[p], heap[p + 1]
            pred = vb < va
            nxt.append((jnp.where(pred, vb, va), jnp.where(pred, gb, ga)))
        heap = nxt
    val, gid = heap[0]
    pred = val < rmin_ref[...]
    rmin_ref[...] = jnp.where(pred, val, rmin_ref[...])
    raux_ref[...] = jnp.where(pred, gid, raux_ref[...])

    @pl.when(j == nc // bc - 1)
    def _():
        rmin = rmin_ref[...]
        gmin = jnp.min(rmin, axis=1, keepdims=True)
        col = (raux_ref[...] * 128
               + jax.lax.broadcasted_iota(jnp.int32, (nt, 128), 1))
        cand = jnp.where(rmin == gmin, col, nc)
        idx_ref[...] = jnp.min(cand, axis=1, keepdims=True)


def _sc_gather(cb_pad, idx_row, nt):
    mesh = plsc.VectorSubcoreMesh(core_axis_name="core",
                                  subcore_axis_name="subcore")
    w = cb_pad.shape[1]

    @pl.kernel(out_type=jax.ShapeDtypeStruct((nt, w), cb_pad.dtype),
               mesh=mesh)
    def gk(cb_hbm, i_hbm, o_hbm):
        def body(i_vmem, o_vmem):
            pltpu.sync_copy(cb_hbm.at[i_vmem.at[0]], o_vmem)

        pltpu.emit_pipeline(
            body,
            grid=(nt // _GW,),
            in_specs=[pl.BlockSpec((1, _GW), index_map=lambda i: (0, i))],
            out_specs=[pl.BlockSpec((_GW, w), index_map=lambda i: (i, 0))],
            core_axis_name=("core", "subcore"),
            dimension_semantics=(pltpu.PARALLEL,),
        )(i_hbm, o_hbm)

    return gk(cb_pad, idx_row)


def _finish_body(x_ref, q_ref, qst_ref, loss_ref):
    x = x_ref[...]
    q = q_ref[:, :x_ref.shape[1]]
    qst_ref[...] = x + (q - x)
    diff = x - q
    msq = jnp.mean(diff * diff)
    loss_ref[...] = jnp.full((1, 1), _BETA * msq + msq, jnp.float32)


def kernel(latent, codebook):
    B, S, D = latent.shape
    nt = B * S
    nc = codebook.shape[0]
    bc = _BC
    flat = latent.reshape(-1, D)
    a = jnp.sum(flat ** 2, axis=1, keepdims=True)
    b = jnp.sum(codebook ** 2, axis=1).reshape(1, nc)
    grid = (nc // bc,)

    idx = pl.pallas_call(
        functools.partial(_argmin_body, nt=nt, nc=nc, bc=bc),
        grid=grid,
        in_specs=[
            pl.BlockSpec((nt, 1), lambda j: (0, 0)),
            pl.BlockSpec((nt, D), lambda j: (0, 0)),
            pl.BlockSpec((bc, D), lambda j: (j, 0)),
            pl.BlockSpec((1, bc), lambda j: (0, j)),
        ],
        out_specs=pl.BlockSpec((nt, 1), lambda j: (0, 0)),
        out_shape=jax.ShapeDtypeStruct((nt, 1), jnp.int32),
        scratch_shapes=[pltpu.VMEM((nt, 128), jnp.float32),
                        pltpu.VMEM((nt, 128), jnp.int32)],
    )(a, 2.0 * flat, codebook, b)

    cb_pad = jnp.pad(codebook, ((0, 0), (0, 128 - D)))
    q = _sc_gather(cb_pad, idx.reshape(1, nt), nt)

    qst, loss = pl.pallas_call(
        _finish_body,
        in_specs=[
            pl.BlockSpec((nt, D), lambda: (0, 0)),
            pl.BlockSpec((nt, 128), lambda: (0, 0)),
        ],
        out_specs=[pl.BlockSpec((nt, D), lambda: (0, 0)),
                   pl.BlockSpec((1, 1), lambda: (0, 0))],
        out_shape=[jax.ShapeDtypeStruct((nt, D), jnp.float32),
                   jax.ShapeDtypeStruct((1, 1), jnp.float32)],
    )(flat, q)

    return (qst.reshape(B, S, D), loss.reshape(()), idx.reshape(nt))
